# baseline (speedup 1.0000x reference)
import functools

import jax
import jax.numpy as jnp
from jax import lax
from jax.experimental import pallas as pl
from jax.experimental.pallas import tpu as pltpu
from jax.experimental.pallas import tpu_sc as plsc

VOCAB = 1000
HIDDEN = 32
NTOK = 4096 * 20

NC, NS = 2, 16
NW = NC * NS
BPW = NTOK // NW
K = 40
NCH = BPW // K


def _table_body(emb_ref, w_ref, b_ref, out_ref):
    out_ref[...] = lax.dot_general(
        emb_ref[...], w_ref[...],
        dimension_numbers=(((1,), (1,)), ((), ())),
        preferred_element_type=jnp.float32,
    ) + b_ref[...]


_table_call = pl.pallas_call(
    _table_body,
    out_shape=jax.ShapeDtypeStruct((VOCAB, VOCAB), jnp.float32),
)


def _sc_gather_body(table_hbm, idx_hbm, out_hbm, idx_v, buf, sem):
    wid = lax.axis_index("s") * NC + lax.axis_index("c")
    pltpu.sync_copy(idx_hbm.at[pl.ds(wid * NCH, NCH)], idx_v)

    def body(c, carry):
        pltpu.async_copy(table_hbm.at[idx_v.at[c]], buf, sem).wait()
        pltpu.sync_copy(buf, out_hbm.at[pl.ds(wid * BPW + c * K, K)])
        return carry

    lax.fori_loop(0, NCH, body, 0)


_sc_gather = functools.partial(
    pl.kernel,
    out_type=jax.ShapeDtypeStruct((NTOK, VOCAB), jnp.float32),
    mesh=plsc.VectorSubcoreMesh(core_axis_name="c", subcore_axis_name="s"),
    scratch_types=[
        pltpu.VMEM((NCH, K), jnp.int32),
        pltpu.VMEM((K, VOCAB), jnp.float32),
        pltpu.SemaphoreType.DMA,
    ],
    compiler_params=pltpu.CompilerParams(use_tc_tiling_on_sc=False),
)(_sc_gather_body)


@jax.jit
def kernel(x, emb_table, W, b):
    table = _table_call(emb_table, W, b.reshape(1, VOCAB))
    idx = x.reshape(NTOK // K, K)
    out = _sc_gather(table, idx)
    return out.reshape(x.shape[0], x.shape[1], VOCAB)


# speedup vs baseline: 1.0432x; 1.0432x over previous
import functools

import jax
import jax.numpy as jnp
from jax import lax
from jax.experimental import pallas as pl
from jax.experimental.pallas import tpu as pltpu
from jax.experimental.pallas import tpu_sc as plsc

VOCAB = 1000
HIDDEN = 32
HPAD = 128
NTOK = 4096 * 20

NC, NS = 2, 16
NW = NC * NS
BPW = NTOK // NW
K = 128
NCH = BPW // K

BLK_T = 512
GRID = NTOK // BLK_T


def _sc_gather_body(emb_hbm, idx_hbm, h_hbm, idx_v, buf, sem):
    wid = lax.axis_index("s") * NC + lax.axis_index("c")
    pltpu.sync_copy(idx_hbm.at[pl.ds(wid * NCH, NCH)], idx_v)

    def body(c, carry):
        pltpu.async_copy(emb_hbm.at[idx_v.at[c]], buf, sem).wait()
        pltpu.sync_copy(buf, h_hbm.at[pl.ds(wid * BPW + c * K, K)])
        return carry

    lax.fori_loop(0, NCH, body, 0)


_sc_gather = functools.partial(
    pl.kernel,
    out_type=jax.ShapeDtypeStruct((NTOK, HPAD), jnp.float32),
    mesh=plsc.VectorSubcoreMesh(core_axis_name="c", subcore_axis_name="s"),
    scratch_types=[
        pltpu.VMEM((NCH, K), jnp.int32),
        pltpu.VMEM((K, HPAD), jnp.float32),
        pltpu.SemaphoreType.DMA,
    ],
    compiler_params=pltpu.CompilerParams(use_tc_tiling_on_sc=False),
)(_sc_gather_body)


def _mm_body(h_ref, w_ref, b_ref, out_ref):
    out_ref[...] = lax.dot_general(
        h_ref[...], w_ref[...],
        dimension_numbers=(((1,), (1,)), ((), ())),
        preferred_element_type=jnp.float32,
    ) + b_ref[...]


_mm_call = pl.pallas_call(
    _mm_body,
    grid=(GRID,),
    in_specs=[
        pl.BlockSpec((BLK_T, HPAD), lambda i: (i, 0)),
        pl.BlockSpec((VOCAB, HPAD), lambda i: (0, 0)),
        pl.BlockSpec((1, VOCAB), lambda i: (0, 0)),
    ],
    out_specs=pl.BlockSpec((BLK_T, VOCAB), lambda i: (i, 0)),
    out_shape=jax.ShapeDtypeStruct((NTOK, VOCAB), jnp.float32),
    compiler_params=pltpu.CompilerParams(
        dimension_semantics=("arbitrary",),
    ),
)


@jax.jit
def kernel(x, emb_table, W, b):
    emb_pad = jnp.zeros((VOCAB, HPAD), jnp.float32).at[:, :HIDDEN].set(emb_table)
    w_pad = jnp.zeros((VOCAB, HPAD), jnp.float32).at[:, :HIDDEN].set(W)
    idx = x.reshape(NTOK // K, K)
    h = _sc_gather(emb_pad, idx)
    out = _mm_call(h, w_pad, b.reshape(1, VOCAB))
    return out.reshape(x.shape[0], x.shape[1], VOCAB)


# speedup vs baseline: 3.4547x; 3.3115x over previous
import functools

import jax
import jax.numpy as jnp
from jax import lax
from jax.experimental import pallas as pl
from jax.experimental.pallas import tpu as pltpu
from jax.experimental.pallas import tpu_sc as plsc

VOCAB = 1000
HIDDEN = 32
HPAD = 128
B, L = 4096, 20
NTOK = B * L

NC, NS = 2, 16
NW = NC * NS
BPW = NTOK // NW
K = 128
NCH = BPW // K

BLK_B = 512
NBB = B // BLK_B


def _sc_gather_body(emb_hbm, idx_hbm, h_hbm, idx_v, buf, sem):
    wid = lax.axis_index("s") * NC + lax.axis_index("c")
    pltpu.sync_copy(idx_hbm.at[pl.ds(wid * NCH, NCH)], idx_v)

    def body(c, carry):
        pltpu.async_copy(emb_hbm.at[idx_v.at[c]], buf, sem).wait()
        pltpu.sync_copy(buf, h_hbm.at[pl.ds(wid * BPW + c * K, K)])
        return carry

    lax.fori_loop(0, NCH, body, 0)


_sc_gather = functools.partial(
    pl.kernel,
    out_type=jax.ShapeDtypeStruct((NTOK, HPAD), jnp.float32),
    mesh=plsc.VectorSubcoreMesh(core_axis_name="c", subcore_axis_name="s"),
    scratch_types=[
        pltpu.VMEM((NCH, K), jnp.int32),
        pltpu.VMEM((K, HPAD), jnp.float32),
        pltpu.SemaphoreType.DMA,
    ],
    compiler_params=pltpu.CompilerParams(use_tc_tiling_on_sc=False),
)(_sc_gather_body)


def _mm_body(w_ref, h_ref, b_ref, out_ref):
    acc = lax.dot_general(
        w_ref[...], h_ref[...],
        dimension_numbers=(((1,), (1,)), ((), ())),
        preferred_element_type=jnp.float32,
    ) + b_ref[...]
    out_ref[...] = acc[None]


_mm_call = pl.pallas_call(
    _mm_body,
    grid=(L, NBB),
    in_specs=[
        pl.BlockSpec((VOCAB, HPAD), lambda l, bb: (0, 0)),
        pl.BlockSpec((BLK_B, HPAD), lambda l, bb: (l * NBB + bb, 0)),
        pl.BlockSpec((VOCAB, 1), lambda l, bb: (0, 0)),
    ],
    out_specs=pl.BlockSpec((1, VOCAB, BLK_B), lambda l, bb: (l, 0, bb)),
    out_shape=jax.ShapeDtypeStruct((L, VOCAB, B), jnp.float32),
    compiler_params=pltpu.CompilerParams(
        dimension_semantics=("arbitrary", "arbitrary"),
    ),
)


@jax.jit
def kernel(x, emb_table, W, b):
    emb_pad = jnp.zeros((VOCAB, HPAD), jnp.float32).at[:, :HIDDEN].set(emb_table)
    w_pad = jnp.zeros((VOCAB, HPAD), jnp.float32).at[:, :HIDDEN].set(W)
    idx = x.T.reshape(NTOK // K, K)
    h = _sc_gather(emb_pad, idx)
    out_phys = _mm_call(w_pad, h, b.reshape(VOCAB, 1))
    return out_phys.transpose(2, 0, 1)


# speedup vs baseline: 4.1973x; 1.2150x over previous
import functools

import jax
import jax.numpy as jnp
from jax import lax
from jax.experimental import pallas as pl
from jax.experimental.pallas import tpu as pltpu
from jax.experimental.pallas import tpu_sc as plsc

VOCAB = 1000
HIDDEN = 32
HPAD = 128
B, L = 4096, 20
NTOK = B * L

NC, NS = 2, 16
NW = NC * NS
BPW = NTOK // NW
K = 128
NCH = BPW // K

BLK_B = 1024
NBB = B // BLK_B


def _sc_gather_body(emb_hbm, idx_hbm, h_hbm, idx_v, buf, sem):
    wid = lax.axis_index("s") * NC + lax.axis_index("c")
    pltpu.sync_copy(idx_hbm.at[pl.ds(wid * NCH, NCH)], idx_v)

    def body(c, carry):
        pltpu.async_copy(emb_hbm.at[idx_v.at[c]], buf, sem).wait()
        pltpu.sync_copy(buf, h_hbm.at[pl.ds(wid * BPW + c * K, K)])
        return carry

    lax.fori_loop(0, NCH, body, 0)


_sc_gather = functools.partial(
    pl.kernel,
    out_type=jax.ShapeDtypeStruct((NTOK, HPAD), jnp.float32),
    mesh=plsc.VectorSubcoreMesh(core_axis_name="c", subcore_axis_name="s"),
    scratch_types=[
        pltpu.VMEM((NCH, K), jnp.int32),
        pltpu.VMEM((K, HPAD), jnp.float32),
        pltpu.SemaphoreType.DMA,
    ],
    compiler_params=pltpu.CompilerParams(use_tc_tiling_on_sc=False),
)(_sc_gather_body)


def _mm_body(w_ref, h_ref, b_ref, out_ref):
    acc = lax.dot_general(
        w_ref[...], h_ref[...],
        dimension_numbers=(((1,), (1,)), ((), ())),
        preferred_element_type=jnp.float32,
    ) + b_ref[...]
    out_ref[...] = acc[None]


_mm_call = pl.pallas_call(
    _mm_body,
    grid=(L, NBB),
    in_specs=[
        pl.BlockSpec((VOCAB, HPAD), lambda l, bb: (0, 0)),
        pl.BlockSpec((BLK_B, HPAD), lambda l, bb: (l * NBB + bb, 0)),
        pl.BlockSpec((VOCAB, 1), lambda l, bb: (0, 0)),
    ],
    out_specs=pl.BlockSpec((1, VOCAB, BLK_B), lambda l, bb: (l, 0, bb)),
    out_shape=jax.ShapeDtypeStruct((L, VOCAB, B), jnp.float32),
    compiler_params=pltpu.CompilerParams(
        dimension_semantics=("arbitrary", "arbitrary"),
    ),
)


@jax.jit
def kernel(x, emb_table, W, b):
    emb_pad = jnp.zeros((VOCAB, HPAD), jnp.float32).at[:, :HIDDEN].set(emb_table)
    w_pad = jnp.zeros((VOCAB, HPAD), jnp.float32).at[:, :HIDDEN].set(W)
    idx = x.T.reshape(NTOK // K, K)
    h = _sc_gather(emb_pad, idx)
    out_phys = _mm_call(w_pad, h, b.reshape(VOCAB, 1))
    return out_phys.transpose(2, 0, 1)


# speedup vs baseline: 4.6241x; 1.1017x over previous
import functools

import jax
import jax.numpy as jnp
from jax import lax
from jax.experimental import pallas as pl
from jax.experimental.pallas import tpu as pltpu
from jax.experimental.pallas import tpu_sc as plsc

VOCAB = 1000
HIDDEN = 32
HPAD = 128
B, L = 4096, 20
NTOK = B * L

NC, NS = 2, 16
NW = NC * NS
BPW = NTOK // NW
K = 128
NCH = BPW // K

BLK_B = 2048
NBB = B // BLK_B


def _sc_gather_body(emb_hbm, idx_hbm, h_hbm, idx_v, buf, sem):
    wid = lax.axis_index("s") * NC + lax.axis_index("c")
    pltpu.sync_copy(idx_hbm.at[pl.ds(wid * NCH, NCH)], idx_v)

    def body(c, carry):
        pltpu.async_copy(emb_hbm.at[idx_v.at[c]], buf, sem).wait()
        pltpu.sync_copy(buf, h_hbm.at[pl.ds(wid * BPW + c * K, K)])
        return carry

    lax.fori_loop(0, NCH, body, 0)


_sc_gather = functools.partial(
    pl.kernel,
    out_type=jax.ShapeDtypeStruct((NTOK, HPAD), jnp.float32),
    mesh=plsc.VectorSubcoreMesh(core_axis_name="c", subcore_axis_name="s"),
    scratch_types=[
        pltpu.VMEM((NCH, K), jnp.int32),
        pltpu.VMEM((K, HPAD), jnp.float32),
        pltpu.SemaphoreType.DMA,
    ],
    compiler_params=pltpu.CompilerParams(use_tc_tiling_on_sc=False),
)(_sc_gather_body)


def _mm_body(w_ref, h_ref, b_ref, out_ref):
    acc = lax.dot_general(
        w_ref[...], h_ref[...],
        dimension_numbers=(((1,), (1,)), ((), ())),
        preferred_element_type=jnp.float32,
    ) + b_ref[...]
    out_ref[...] = acc[None]


_mm_call = pl.pallas_call(
    _mm_body,
    grid=(L, NBB),
    in_specs=[
        pl.BlockSpec((VOCAB, HPAD), lambda l, bb: (0, 0)),
        pl.BlockSpec((BLK_B, HPAD), lambda l, bb: (l * NBB + bb, 0)),
        pl.BlockSpec((VOCAB, 1), lambda l, bb: (0, 0)),
    ],
    out_specs=pl.BlockSpec((1, VOCAB, BLK_B), lambda l, bb: (l, 0, bb)),
    out_shape=jax.ShapeDtypeStruct((L, VOCAB, B), jnp.float32),
    compiler_params=pltpu.CompilerParams(
        dimension_semantics=("arbitrary", "arbitrary"),
    ),
)


@jax.jit
def kernel(x, emb_table, W, b):
    emb_pad = jnp.zeros((VOCAB, HPAD), jnp.float32).at[:, :HIDDEN].set(emb_table)
    w_pad = jnp.zeros((VOCAB, HPAD), jnp.float32).at[:, :HIDDEN].set(W)
    idx = x.T.reshape(NTOK // K, K)
    h = _sc_gather(emb_pad, idx)
    out_phys = _mm_call(w_pad, h, b.reshape(VOCAB, 1))
    return out_phys.transpose(2, 0, 1)
